# baseline (speedup 1.0000x reference)
import functools

import jax
import jax.numpy as jnp
from jax import lax
from jax.experimental import pallas as pl
from jax.experimental.pallas import tpu as pltpu
from jax.experimental.pallas import tpu_sc as plsc

N = 10000
E = 160000
D = 128
DH = 64

NC = 2
NS = 16
G = 128
BI = 16
BO = 5
NB = 3
C = BI * BO
EPT = C * G
E_PAD = NS * EPT
PAD = E_PAD - E

RPT = 632
ACC_ROWS = NS * RPT
LANES = 16

_MESH = plsc.VectorSubcoreMesh(
    core_axis_name="c", subcore_axis_name="s", num_cores=NC, num_subcores=NS
)


def _zero_vmem_rows(ref, nrows, ncols):
    zvec = jnp.zeros((LANES,), jnp.float32)

    def row(i, _):
        def col(j, _):
            ref[i, pl.ds(j * LANES, LANES)] = zvec
            return 0

        return lax.fori_loop(0, ncols // LANES, col, 0)

    lax.fori_loop(0, nrows, row, 0)


def _zero_acc_slice(src_rows, acc, base):
    for k in range(RPT // G):
        pltpu.sync_copy(src_rows, acc.at[pl.ds(base + k * G, G)])
    rem = RPT % G
    if rem:
        pltpu.sync_copy(
            src_rows.at[pl.ds(0, rem)],
            acc.at[pl.ds(base + (RPT // G) * G, rem)],
        )


def _sc_layer_body(h_hbm, src_hbm, dst_hbm, agg_hbm,
                   table, acc, sidx, didx, *bufs):
    c = lax.axis_index("c")
    s = lax.axis_index("s")
    base = s * RPT
    last = NS - 1

    rows = bufs[:NB]
    semg = bufs[NB:2 * NB]
    sems = bufs[2 * NB:3 * NB]

    for p in range(D // DH):
        col = p * DH

        @pl.when(s < last)
        def _():
            pltpu.sync_copy(
                h_hbm.at[pl.ds(s * RPT, RPT), pl.ds(col, DH)],
                table.at[pl.ds(s * RPT, RPT)],
            )

        @pl.when(s == last)
        def _():
            pltpu.sync_copy(
                h_hbm.at[pl.ds(last * RPT, N - last * RPT), pl.ds(col, DH)],
                table.at[pl.ds(last * RPT, N - last * RPT)],
            )

        _zero_vmem_rows(rows[0], G, DH)
        _zero_acc_slice(rows[0], acc, base)
        plsc.subcore_barrier()

        def block(k, _):
            pltpu.sync_copy(src_hbm.at[c, s, pl.ds(k * BI, BI)], sidx)
            pltpu.sync_copy(dst_hbm.at[c, s, pl.ds(k * BI, BI)], didx)
            gd = [None] * NB
            sd = [None] * NB
            for j in range(BI):
                b = j % NB
                if sd[b] is not None:
                    sd[b].wait()
                gd[b] = pltpu.async_copy(table.at[sidx.at[j]], rows[b], semg[b])
                gd[b].wait()
                sd[b] = pltpu.async_copy(
                    rows[b], acc.at[didx.at[j]], sems[b], add=True
                )
            for b in range(min(NB, BI)):
                if sd[b] is not None:
                    sd[b].wait()
            return 0

        lax.fori_loop(0, BO, block, 0)
        plsc.subcore_barrier()

        pltpu.sync_copy(
            acc.at[pl.ds(base, RPT)],
            agg_hbm.at[c, pl.ds(base, RPT), pl.ds(col, DH)],
        )


def _sc_deg_body(dst_hbm, deg_hbm, dacc, didx, ones):
    c = lax.axis_index("c")
    s = lax.axis_index("s")
    base = s * RPT

    _zero_vmem_rows(ones, G, LANES)
    _zero_acc_slice(ones, dacc, base)
    onehot = jnp.where(
        lax.iota(jnp.int32, LANES) == 0, jnp.float32(1.0), jnp.float32(0.0)
    )

    def set_onehot(i, _):
        ones[i, :] = onehot
        return 0

    lax.fori_loop(0, G, set_onehot, 0)
    plsc.subcore_barrier()

    def block(b, _):
        pltpu.sync_copy(dst_hbm.at[c, s, pl.ds(b * BI, BI)], didx)

        def step(g, _):
            pltpu.sync_copy(ones, dacc.at[didx.at[g]], add=True)
            return 0

        return lax.fori_loop(0, BI, step, 0)

    lax.fori_loop(0, BO, block, 0)
    plsc.subcore_barrier()

    pltpu.sync_copy(dacc.at[pl.ds(base, RPT)], deg_hbm.at[c, pl.ds(base, RPT)])


_sc_layer = pl.kernel(
    _sc_layer_body,
    out_type=jax.ShapeDtypeStruct((NC, ACC_ROWS, D), jnp.float32),
    mesh=_MESH,
    scratch_types=(
        [
            pltpu.VMEM_SHARED((ACC_ROWS, DH), jnp.float32),
            pltpu.VMEM_SHARED((ACC_ROWS, DH), jnp.float32),
            pltpu.VMEM((BI, G), jnp.int32),
            pltpu.VMEM((BI, G), jnp.int32),
        ]
        + [pltpu.VMEM((G, DH), jnp.float32) for _ in range(NB)]
        + [pltpu.SemaphoreType.DMA for _ in range(2 * NB)]
    ),
    compiler_params=pltpu.CompilerParams(
        use_tc_tiling_on_sc=False, skip_device_barrier=True
    ),
)

_sc_deg = pl.kernel(
    _sc_deg_body,
    out_type=jax.ShapeDtypeStruct((NC, ACC_ROWS, LANES), jnp.float32),
    mesh=_MESH,
    scratch_types=[
        pltpu.VMEM_SHARED((ACC_ROWS, LANES), jnp.float32),
        pltpu.VMEM((BI, G), jnp.int32),
        pltpu.VMEM((G, LANES), jnp.float32),
    ],
    compiler_params=pltpu.CompilerParams(
        use_tc_tiling_on_sc=False, skip_device_barrier=True
    ),
)


_BM = 1000
_GRID = N // _BM


def _enc_body(x_ref, w_ref, b_ref, o_ref):
    h = jnp.dot(x_ref[:, :], w_ref[:, :], preferred_element_type=jnp.float32)
    o_ref[:, :] = jax.nn.gelu(h + b_ref[:, :])


_encoder = pl.pallas_call(
    _enc_body,
    grid=(_GRID,),
    in_specs=[
        pl.BlockSpec((_BM, D), lambda i: (i, 0)),
        pl.BlockSpec((D, D), lambda i: (0, 0)),
        pl.BlockSpec((1, D), lambda i: (0, 0)),
    ],
    out_specs=pl.BlockSpec((_BM, D), lambda i: (i, 0)),
    out_shape=jax.ShapeDtypeStruct((N, D), jnp.float32),
)


def _norm2(a0_ref, d0_ref, a1_ref, d1_ref):
    inv0 = 1.0 / jnp.maximum(d0_ref[:, 0:1], 1.0)
    inv1 = 1.0 / jnp.maximum(d1_ref[:, 0:1], 1.0)
    return a0_ref[:, :] * inv0, a1_ref[:, :] * inv1


def _comb_body(a0_ref, d0_ref, a1_ref, d1_ref, w0_ref, w1_ref, o_ref):
    m0, m1 = _norm2(a0_ref, d0_ref, a1_ref, d1_ref)
    o_ref[:, :] = (
        jnp.dot(m0, w0_ref[:, :], preferred_element_type=jnp.float32)
        + jnp.dot(m1, w1_ref[:, :], preferred_element_type=jnp.float32)
    )


def _comb_out_body(a0_ref, d0_ref, a1_ref, d1_ref, w0_ref, w1_ref,
                   wo_ref, bo_ref, o_ref):
    m0, m1 = _norm2(a0_ref, d0_ref, a1_ref, d1_ref)
    h = (
        jnp.dot(m0, w0_ref[:, :], preferred_element_type=jnp.float32)
        + jnp.dot(m1, w1_ref[:, :], preferred_element_type=jnp.float32)
    )
    o_ref[:, :] = (
        jnp.dot(h, wo_ref[:, :], preferred_element_type=jnp.float32)
        + bo_ref[:, :]
    )


_AGG_SPEC = pl.BlockSpec((_BM, D), lambda i: (i, 0))
_DEG_SPEC = pl.BlockSpec((_BM, LANES), lambda i: (i, 0))
_W_SPEC = pl.BlockSpec((D, D), lambda i: (0, 0))

_combine = pl.pallas_call(
    _comb_body,
    grid=(_GRID,),
    in_specs=[_AGG_SPEC, _DEG_SPEC, _AGG_SPEC, _DEG_SPEC, _W_SPEC, _W_SPEC],
    out_specs=pl.BlockSpec((_BM, D), lambda i: (i, 0)),
    out_shape=jax.ShapeDtypeStruct((N, D), jnp.float32),
)

_combine_out = pl.pallas_call(
    _comb_out_body,
    grid=(_GRID,),
    in_specs=[_AGG_SPEC, _DEG_SPEC, _AGG_SPEC, _DEG_SPEC, _W_SPEC, _W_SPEC,
              _W_SPEC, pl.BlockSpec((1, D), lambda i: (0, 0))],
    out_specs=pl.BlockSpec((_BM, D), lambda i: (i, 0)),
    out_shape=jax.ShapeDtypeStruct((N, D), jnp.float32),
)


def _prep_edges(ei):
    src = jnp.concatenate([ei[0], jnp.zeros((PAD,), jnp.int32)])
    dst = jnp.concatenate([ei[1], jnp.full((PAD,), N, jnp.int32)])
    return src.reshape(NS, C, G), dst.reshape(NS, C, G)


def kernel(x, edge_index_rel0, edge_index_rel1, W_enc, b_enc,
           W00, W01, W10, W11, W_out, b_out):
    s0, d0 = _prep_edges(edge_index_rel0)
    s1, d1 = _prep_edges(edge_index_rel1)
    src_all = jnp.stack([s0, s1])
    dst_all = jnp.stack([d0, d1])

    deg = _sc_deg(dst_all)
    h = _encoder(x, W_enc, b_enc.reshape(1, D))
    agg = _sc_layer(h, src_all, dst_all)
    h = _combine(agg[0], deg[0], agg[1], deg[1], W00, W01)
    agg2 = _sc_layer(h, src_all, dst_all)
    return _combine_out(agg2[0], deg[0], agg2[1], deg[1], W10, W11,
                        W_out, b_out.reshape(1, D))


# speedup vs baseline: 1.0167x; 1.0167x over previous
import functools

import jax
import jax.numpy as jnp
from jax import lax
from jax.experimental import pallas as pl
from jax.experimental.pallas import tpu as pltpu
from jax.experimental.pallas import tpu_sc as plsc

N = 10000
E = 160000
D = 128
DH = 64

NC = 2
NS = 16
G = 128
BI = 16
BO = 5
NB = 3
C = BI * BO
EPT = C * G
E_PAD = NS * EPT
PAD = E_PAD - E

RPT = 632
ACC_ROWS = NS * RPT
LANES = 16

_MESH = plsc.VectorSubcoreMesh(
    core_axis_name="c", subcore_axis_name="s", num_cores=NC, num_subcores=NS
)


def _zero_vmem_rows(ref, nrows, ncols):
    zvec = jnp.zeros((LANES,), jnp.float32)

    def row(i, _):
        def col(j, _):
            ref[i, pl.ds(j * LANES, LANES)] = zvec
            return 0

        return lax.fori_loop(0, ncols // LANES, col, 0)

    lax.fori_loop(0, nrows, row, 0)


def _zero_acc_slice(src_rows, acc, base):
    for k in range(RPT // G):
        pltpu.sync_copy(src_rows, acc.at[pl.ds(base + k * G, G)])
    rem = RPT % G
    if rem:
        pltpu.sync_copy(
            src_rows.at[pl.ds(0, rem)],
            acc.at[pl.ds(base + (RPT // G) * G, rem)],
        )


def _sc_layer_body(h_hbm, src_hbm, dst_hbm, agg_hbm,
                   table, acc, sidx, didx, *bufs):
    c = lax.axis_index("c")
    s = lax.axis_index("s")
    base = s * RPT
    last = NS - 1

    rows = bufs[:NB]
    semg = bufs[NB:2 * NB]
    sems = bufs[2 * NB:3 * NB]

    for p in range(D // DH):
        col = p * DH

        @pl.when(s < last)
        def _():
            pltpu.async_copy(
                h_hbm.at[pl.ds(s * RPT, RPT), pl.ds(col, DH)],
                table.at[pl.ds(s * RPT, RPT)],
                semg[0],
            )

        @pl.when(s == last)
        def _():
            pltpu.async_copy(
                h_hbm.at[pl.ds(last * RPT, N - last * RPT), pl.ds(col, DH)],
                table.at[pl.ds(last * RPT, N - last * RPT)],
                semg[0],
            )

        _zero_vmem_rows(rows[0], G, DH)
        _zero_acc_slice(rows[0], acc, base)

        @pl.when(s < last)
        def _():
            pltpu.make_async_copy(
                h_hbm.at[pl.ds(s * RPT, RPT), pl.ds(col, DH)],
                table.at[pl.ds(s * RPT, RPT)],
                semg[0],
            ).wait()

        @pl.when(s == last)
        def _():
            pltpu.make_async_copy(
                h_hbm.at[pl.ds(last * RPT, N - last * RPT), pl.ds(col, DH)],
                table.at[pl.ds(last * RPT, N - last * RPT)],
                semg[0],
            ).wait()

        plsc.subcore_barrier()

        def block(k, _):
            pltpu.sync_copy(src_hbm.at[c, s, pl.ds(k * BI, BI)], sidx)
            pltpu.sync_copy(dst_hbm.at[c, s, pl.ds(k * BI, BI)], didx)
            gd = [None] * NB
            sd = [None] * NB
            for j in range(BI):
                b = j % NB
                if sd[b] is not None:
                    sd[b].wait()
                gd[b] = pltpu.async_copy(table.at[sidx.at[j]], rows[b], semg[b])
                gd[b].wait()
                sd[b] = pltpu.async_copy(
                    rows[b], acc.at[didx.at[j]], sems[b], add=True
                )
            for b in range(min(NB, BI)):
                if sd[b] is not None:
                    sd[b].wait()
            return 0

        lax.fori_loop(0, BO, block, 0)
        plsc.subcore_barrier()

        pltpu.sync_copy(
            acc.at[pl.ds(base, RPT)],
            agg_hbm.at[c, pl.ds(base, RPT), pl.ds(col, DH)],
        )


def _sc_deg_body(dst_hbm, deg_hbm, dacc, didx, ones):
    c = lax.axis_index("c")
    s = lax.axis_index("s")
    base = s * RPT

    _zero_vmem_rows(ones, G, LANES)
    _zero_acc_slice(ones, dacc, base)
    onehot = jnp.where(
        lax.iota(jnp.int32, LANES) == 0, jnp.float32(1.0), jnp.float32(0.0)
    )

    def set_onehot(i, _):
        ones[i, :] = onehot
        return 0

    lax.fori_loop(0, G, set_onehot, 0)
    plsc.subcore_barrier()

    def block(b, _):
        pltpu.sync_copy(dst_hbm.at[c, s, pl.ds(b * BI, BI)], didx)

        def step(g, _):
            pltpu.sync_copy(ones, dacc.at[didx.at[g]], add=True)
            return 0

        return lax.fori_loop(0, BI, step, 0)

    lax.fori_loop(0, BO, block, 0)
    plsc.subcore_barrier()

    pltpu.sync_copy(dacc.at[pl.ds(base, RPT)], deg_hbm.at[c, pl.ds(base, RPT)])


_sc_layer = pl.kernel(
    _sc_layer_body,
    out_type=jax.ShapeDtypeStruct((NC, ACC_ROWS, D), jnp.float32),
    mesh=_MESH,
    scratch_types=(
        [
            pltpu.VMEM_SHARED((ACC_ROWS, DH), jnp.float32),
            pltpu.VMEM_SHARED((ACC_ROWS, DH), jnp.float32),
            pltpu.VMEM((BI, G), jnp.int32),
            pltpu.VMEM((BI, G), jnp.int32),
        ]
        + [pltpu.VMEM((G, DH), jnp.float32) for _ in range(NB)]
        + [pltpu.SemaphoreType.DMA for _ in range(2 * NB)]
    ),
    compiler_params=pltpu.CompilerParams(use_tc_tiling_on_sc=False),
)

_sc_deg = pl.kernel(
    _sc_deg_body,
    out_type=jax.ShapeDtypeStruct((NC, ACC_ROWS, LANES), jnp.float32),
    mesh=_MESH,
    scratch_types=[
        pltpu.VMEM_SHARED((ACC_ROWS, LANES), jnp.float32),
        pltpu.VMEM((BI, G), jnp.int32),
        pltpu.VMEM((G, LANES), jnp.float32),
    ],
    compiler_params=pltpu.CompilerParams(use_tc_tiling_on_sc=False),
)


_BM = 1000
_GRID = N // _BM


def _enc_body(x_ref, w_ref, b_ref, o_ref):
    h = jnp.dot(x_ref[:, :], w_ref[:, :], preferred_element_type=jnp.float32)
    o_ref[:, :] = jax.nn.gelu(h + b_ref[:, :])


_encoder = pl.pallas_call(
    _enc_body,
    grid=(_GRID,),
    in_specs=[
        pl.BlockSpec((_BM, D), lambda i: (i, 0)),
        pl.BlockSpec((D, D), lambda i: (0, 0)),
        pl.BlockSpec((1, D), lambda i: (0, 0)),
    ],
    out_specs=pl.BlockSpec((_BM, D), lambda i: (i, 0)),
    out_shape=jax.ShapeDtypeStruct((N, D), jnp.float32),
)


def _norm2(a0_ref, d0_ref, a1_ref, d1_ref):
    inv0 = 1.0 / jnp.maximum(d0_ref[:, 0:1], 1.0)
    inv1 = 1.0 / jnp.maximum(d1_ref[:, 0:1], 1.0)
    return a0_ref[:, :] * inv0, a1_ref[:, :] * inv1


def _comb_body(a0_ref, d0_ref, a1_ref, d1_ref, w0_ref, w1_ref, o_ref):
    m0, m1 = _norm2(a0_ref, d0_ref, a1_ref, d1_ref)
    o_ref[:, :] = (
        jnp.dot(m0, w0_ref[:, :], preferred_element_type=jnp.float32)
        + jnp.dot(m1, w1_ref[:, :], preferred_element_type=jnp.float32)
    )


def _comb_out_body(a0_ref, d0_ref, a1_ref, d1_ref, w0_ref, w1_ref,
                   wo_ref, bo_ref, o_ref):
    m0, m1 = _norm2(a0_ref, d0_ref, a1_ref, d1_ref)
    h = (
        jnp.dot(m0, w0_ref[:, :], preferred_element_type=jnp.float32)
        + jnp.dot(m1, w1_ref[:, :], preferred_element_type=jnp.float32)
    )
    o_ref[:, :] = (
        jnp.dot(h, wo_ref[:, :], preferred_element_type=jnp.float32)
        + bo_ref[:, :]
    )


_AGG_SPEC = pl.BlockSpec((_BM, D), lambda i: (i, 0))
_DEG_SPEC = pl.BlockSpec((_BM, LANES), lambda i: (i, 0))
_W_SPEC = pl.BlockSpec((D, D), lambda i: (0, 0))

_combine = pl.pallas_call(
    _comb_body,
    grid=(_GRID,),
    in_specs=[_AGG_SPEC, _DEG_SPEC, _AGG_SPEC, _DEG_SPEC, _W_SPEC, _W_SPEC],
    out_specs=pl.BlockSpec((_BM, D), lambda i: (i, 0)),
    out_shape=jax.ShapeDtypeStruct((N, D), jnp.float32),
)

_combine_out = pl.pallas_call(
    _comb_out_body,
    grid=(_GRID,),
    in_specs=[_AGG_SPEC, _DEG_SPEC, _AGG_SPEC, _DEG_SPEC, _W_SPEC, _W_SPEC,
              _W_SPEC, pl.BlockSpec((1, D), lambda i: (0, 0))],
    out_specs=pl.BlockSpec((_BM, D), lambda i: (i, 0)),
    out_shape=jax.ShapeDtypeStruct((N, D), jnp.float32),
)


def _prep_edges(ei):
    src = jnp.concatenate([ei[0], jnp.zeros((PAD,), jnp.int32)])
    dst = jnp.concatenate([ei[1], jnp.full((PAD,), N, jnp.int32)])
    return src.reshape(NS, C, G), dst.reshape(NS, C, G)


def kernel(x, edge_index_rel0, edge_index_rel1, W_enc, b_enc,
           W00, W01, W10, W11, W_out, b_out):
    s0, d0 = _prep_edges(edge_index_rel0)
    s1, d1 = _prep_edges(edge_index_rel1)
    src_all = jnp.stack([s0, s1])
    dst_all = jnp.stack([d0, d1])

    deg = _sc_deg(dst_all)
    h = _encoder(x, W_enc, b_enc.reshape(1, D))
    agg = _sc_layer(h, src_all, dst_all)
    h = _combine(agg[0], deg[0], agg[1], deg[1], W00, W01)
    agg2 = _sc_layer(h, src_all, dst_all)
    return _combine_out(agg2[0], deg[0], agg2[1], deg[1], W10, W11,
                        W_out, b_out.reshape(1, D))


# speedup vs baseline: 1.0340x; 1.0170x over previous
import functools

import jax
import jax.numpy as jnp
from jax import lax
from jax.experimental import pallas as pl
from jax.experimental.pallas import tpu as pltpu
from jax.experimental.pallas import tpu_sc as plsc

N = 10000
E = 160000
D = 128
DH = 64

NC = 2
NS = 16
G = 128
BI = 20
BO = 4
NB = 3
C = BI * BO
EPT = C * G
E_PAD = NS * EPT
PAD = E_PAD - E

RPT = 632
ACC_ROWS = NS * RPT
LANES = 16

_MESH = plsc.VectorSubcoreMesh(
    core_axis_name="c", subcore_axis_name="s", num_cores=NC, num_subcores=NS
)


def _zero_vmem_rows(ref, nrows, ncols):
    zvec = jnp.zeros((LANES,), jnp.float32)

    def row(i, _):
        def col(j, _):
            ref[i, pl.ds(j * LANES, LANES)] = zvec
            return 0

        return lax.fori_loop(0, ncols // LANES, col, 0)

    lax.fori_loop(0, nrows, row, 0)


def _zero_acc_slice(src_rows, acc, base):
    for k in range(RPT // G):
        pltpu.sync_copy(src_rows, acc.at[pl.ds(base + k * G, G)])
    rem = RPT % G
    if rem:
        pltpu.sync_copy(
            src_rows.at[pl.ds(0, rem)],
            acc.at[pl.ds(base + (RPT // G) * G, rem)],
        )


def _sc_layer_body(h_hbm, src_hbm, dst_hbm, agg_hbm,
                   table, acc, sidx, didx, *bufs):
    c = lax.axis_index("c")
    s = lax.axis_index("s")
    base = s * RPT
    last = NS - 1

    rows = bufs[:NB]
    semg = bufs[NB:2 * NB]
    sems = bufs[2 * NB:3 * NB]

    for p in range(D // DH):
        col = p * DH

        @pl.when(s < last)
        def _():
            pltpu.async_copy(
                h_hbm.at[pl.ds(s * RPT, RPT), pl.ds(col, DH)],
                table.at[pl.ds(s * RPT, RPT)],
                semg[0],
            )

        @pl.when(s == last)
        def _():
            pltpu.async_copy(
                h_hbm.at[pl.ds(last * RPT, N - last * RPT), pl.ds(col, DH)],
                table.at[pl.ds(last * RPT, N - last * RPT)],
                semg[0],
            )

        _zero_vmem_rows(rows[0], G, DH)
        _zero_acc_slice(rows[0], acc, base)

        @pl.when(s < last)
        def _():
            pltpu.make_async_copy(
                h_hbm.at[pl.ds(s * RPT, RPT), pl.ds(col, DH)],
                table.at[pl.ds(s * RPT, RPT)],
                semg[0],
            ).wait()

        @pl.when(s == last)
        def _():
            pltpu.make_async_copy(
                h_hbm.at[pl.ds(last * RPT, N - last * RPT), pl.ds(col, DH)],
                table.at[pl.ds(last * RPT, N - last * RPT)],
                semg[0],
            ).wait()

        plsc.subcore_barrier()

        def block(k, _):
            pltpu.sync_copy(src_hbm.at[c, s, pl.ds(k * BI, BI)], sidx)
            pltpu.sync_copy(dst_hbm.at[c, s, pl.ds(k * BI, BI)], didx)
            gd = [None] * NB
            sd = [None] * NB
            for j in range(BI):
                b = j % NB
                if sd[b] is not None:
                    sd[b].wait()
                gd[b] = pltpu.async_copy(table.at[sidx.at[j]], rows[b], semg[b])
                gd[b].wait()
                sd[b] = pltpu.async_copy(
                    rows[b], acc.at[didx.at[j]], sems[b], add=True
                )
            for b in range(min(NB, BI)):
                if sd[b] is not None:
                    sd[b].wait()
            return 0

        lax.fori_loop(0, BO, block, 0)
        plsc.subcore_barrier()

        pltpu.sync_copy(
            acc.at[pl.ds(base, RPT)],
            agg_hbm.at[c, pl.ds(base, RPT), pl.ds(col, DH)],
        )


def _sc_deg_body(dst_hbm, deg_hbm, dacc, didx, ones):
    c = lax.axis_index("c")
    s = lax.axis_index("s")
    base = s * RPT

    _zero_vmem_rows(ones, G, LANES)
    _zero_acc_slice(ones, dacc, base)
    onehot = jnp.where(
        lax.iota(jnp.int32, LANES) == 0, jnp.float32(1.0), jnp.float32(0.0)
    )

    def set_onehot(i, _):
        ones[i, :] = onehot
        return 0

    lax.fori_loop(0, G, set_onehot, 0)
    plsc.subcore_barrier()

    def block(b, _):
        pltpu.sync_copy(dst_hbm.at[c, s, pl.ds(b * BI, BI)], didx)

        def step(g, _):
            pltpu.sync_copy(ones, dacc.at[didx.at[g]], add=True)
            return 0

        return lax.fori_loop(0, BI, step, 0)

    lax.fori_loop(0, BO, block, 0)
    plsc.subcore_barrier()

    pltpu.sync_copy(dacc.at[pl.ds(base, RPT)], deg_hbm.at[c, pl.ds(base, RPT)])


_sc_layer = pl.kernel(
    _sc_layer_body,
    out_type=jax.ShapeDtypeStruct((NC, ACC_ROWS, D), jnp.float32),
    mesh=_MESH,
    scratch_types=(
        [
            pltpu.VMEM_SHARED((ACC_ROWS, DH), jnp.float32),
            pltpu.VMEM_SHARED((ACC_ROWS, DH), jnp.float32),
            pltpu.VMEM((BI, G), jnp.int32),
            pltpu.VMEM((BI, G), jnp.int32),
        ]
        + [pltpu.VMEM((G, DH), jnp.float32) for _ in range(NB)]
        + [pltpu.SemaphoreType.DMA for _ in range(2 * NB)]
    ),
    compiler_params=pltpu.CompilerParams(use_tc_tiling_on_sc=False),
)

_sc_deg = pl.kernel(
    _sc_deg_body,
    out_type=jax.ShapeDtypeStruct((NC, ACC_ROWS, LANES), jnp.float32),
    mesh=_MESH,
    scratch_types=[
        pltpu.VMEM_SHARED((ACC_ROWS, LANES), jnp.float32),
        pltpu.VMEM((BI, G), jnp.int32),
        pltpu.VMEM((G, LANES), jnp.float32),
    ],
    compiler_params=pltpu.CompilerParams(use_tc_tiling_on_sc=False),
)


_BM = 1000
_GRID = N // _BM


def _enc_body(x_ref, w_ref, b_ref, o_ref):
    h = jnp.dot(x_ref[:, :], w_ref[:, :], preferred_element_type=jnp.float32)
    o_ref[:, :] = jax.nn.gelu(h + b_ref[:, :])


_encoder = pl.pallas_call(
    _enc_body,
    grid=(_GRID,),
    in_specs=[
        pl.BlockSpec((_BM, D), lambda i: (i, 0)),
        pl.BlockSpec((D, D), lambda i: (0, 0)),
        pl.BlockSpec((1, D), lambda i: (0, 0)),
    ],
    out_specs=pl.BlockSpec((_BM, D), lambda i: (i, 0)),
    out_shape=jax.ShapeDtypeStruct((N, D), jnp.float32),
)


def _norm2(a0_ref, d0_ref, a1_ref, d1_ref):
    inv0 = 1.0 / jnp.maximum(d0_ref[:, 0:1], 1.0)
    inv1 = 1.0 / jnp.maximum(d1_ref[:, 0:1], 1.0)
    return a0_ref[:, :] * inv0, a1_ref[:, :] * inv1


def _comb_body(a0_ref, d0_ref, a1_ref, d1_ref, w0_ref, w1_ref, o_ref):
    m0, m1 = _norm2(a0_ref, d0_ref, a1_ref, d1_ref)
    o_ref[:, :] = (
        jnp.dot(m0, w0_ref[:, :], preferred_element_type=jnp.float32)
        + jnp.dot(m1, w1_ref[:, :], preferred_element_type=jnp.float32)
    )


def _comb_out_body(a0_ref, d0_ref, a1_ref, d1_ref, w0_ref, w1_ref,
                   wo_ref, bo_ref, o_ref):
    m0, m1 = _norm2(a0_ref, d0_ref, a1_ref, d1_ref)
    h = (
        jnp.dot(m0, w0_ref[:, :], preferred_element_type=jnp.float32)
        + jnp.dot(m1, w1_ref[:, :], preferred_element_type=jnp.float32)
    )
    o_ref[:, :] = (
        jnp.dot(h, wo_ref[:, :], preferred_element_type=jnp.float32)
        + bo_ref[:, :]
    )


_AGG_SPEC = pl.BlockSpec((_BM, D), lambda i: (i, 0))
_DEG_SPEC = pl.BlockSpec((_BM, LANES), lambda i: (i, 0))
_W_SPEC = pl.BlockSpec((D, D), lambda i: (0, 0))

_combine = pl.pallas_call(
    _comb_body,
    grid=(_GRID,),
    in_specs=[_AGG_SPEC, _DEG_SPEC, _AGG_SPEC, _DEG_SPEC, _W_SPEC, _W_SPEC],
    out_specs=pl.BlockSpec((_BM, D), lambda i: (i, 0)),
    out_shape=jax.ShapeDtypeStruct((N, D), jnp.float32),
)

_combine_out = pl.pallas_call(
    _comb_out_body,
    grid=(_GRID,),
    in_specs=[_AGG_SPEC, _DEG_SPEC, _AGG_SPEC, _DEG_SPEC, _W_SPEC, _W_SPEC,
              _W_SPEC, pl.BlockSpec((1, D), lambda i: (0, 0))],
    out_specs=pl.BlockSpec((_BM, D), lambda i: (i, 0)),
    out_shape=jax.ShapeDtypeStruct((N, D), jnp.float32),
)


def _prep_edges(ei):
    src = jnp.concatenate([ei[0], jnp.zeros((PAD,), jnp.int32)])
    dst = jnp.concatenate([ei[1], jnp.full((PAD,), N, jnp.int32)])
    return src.reshape(NS, C, G), dst.reshape(NS, C, G)


def kernel(x, edge_index_rel0, edge_index_rel1, W_enc, b_enc,
           W00, W01, W10, W11, W_out, b_out):
    s0, d0 = _prep_edges(edge_index_rel0)
    s1, d1 = _prep_edges(edge_index_rel1)
    src_all = jnp.stack([s0, s1])
    dst_all = jnp.stack([d0, d1])

    deg = _sc_deg(dst_all)
    h = _encoder(x, W_enc, b_enc.reshape(1, D))
    agg = _sc_layer(h, src_all, dst_all)
    h = _combine(agg[0], deg[0], agg[1], deg[1], W00, W01)
    agg2 = _sc_layer(h, src_all, dst_all)
    return _combine_out(agg2[0], deg[0], agg2[1], deg[1], W10, W11,
                        W_out, b_out.reshape(1, D))
